# Initial kernel scaffold; baseline (speedup 1.0000x reference)
import jax
import jax.numpy as jnp
from jax.experimental import pallas as pl


def kernel(x, emb_table):
    raise NotImplementedError("write your pallas kernel here")


# speedup vs baseline: 2.1186x; 2.1186x over previous
import jax
import jax.numpy as jnp
from jax.experimental import pallas as pl

BLK = 256


def _body(x_ref, e_ref, o_ref):
    o_ref[...] = x_ref[...] + e_ref[...][:, None, :]


def kernel(x, emb_table):
    M, B, D = x.shape
    return pl.pallas_call(
        _body,
        grid=(M // BLK,),
        in_specs=[
            pl.BlockSpec((BLK, B, D), lambda i: (i, 0, 0)),
            pl.BlockSpec((BLK, D), lambda i: (i, 0)),
        ],
        out_specs=pl.BlockSpec((BLK, B, D), lambda i: (i, 0, 0)),
        out_shape=jax.ShapeDtypeStruct((M, B, D), x.dtype),
    )(x, emb_table)


# speedup vs baseline: 2.2755x; 1.0741x over previous
import jax
import jax.numpy as jnp
from jax.experimental import pallas as pl

BLK = 512


def _body(x_ref, e_ref, o_ref):
    o_ref[...] = x_ref[...] + e_ref[...][:, None, :]


def kernel(x, emb_table):
    M, B, D = x.shape
    return pl.pallas_call(
        _body,
        grid=(M // BLK,),
        in_specs=[
            pl.BlockSpec((BLK, B, D), lambda i: (i, 0, 0)),
            pl.BlockSpec((BLK, D), lambda i: (i, 0)),
        ],
        out_specs=pl.BlockSpec((BLK, B, D), lambda i: (i, 0, 0)),
        out_shape=jax.ShapeDtypeStruct((M, B, D), x.dtype),
    )(x, emb_table)


# speedup vs baseline: 2.3566x; 1.0356x over previous
import jax
import jax.numpy as jnp
from jax.experimental import pallas as pl

BLK = 1024


def _body(x_ref, e_ref, o_ref):
    o_ref[...] = x_ref[...] + e_ref[...][:, None, :]


def kernel(x, emb_table):
    M, B, D = x.shape
    return pl.pallas_call(
        _body,
        grid=(M // BLK,),
        in_specs=[
            pl.BlockSpec((BLK, B, D), lambda i: (i, 0, 0)),
            pl.BlockSpec((BLK, D), lambda i: (i, 0)),
        ],
        out_specs=pl.BlockSpec((BLK, B, D), lambda i: (i, 0, 0)),
        out_shape=jax.ShapeDtypeStruct((M, B, D), x.dtype),
    )(x, emb_table)
